# baseline (speedup 1.0000x reference)
import jax
import jax.numpy as jnp
from jax.experimental import pallas as pl

_B, _K, _D = 64, 32, 64
_E = _B * _K
_NEG = -1e30
_TA = 8
_PACK = 4


def _relu(v):
    return jnp.maximum(v, 0.0)


def _dot_t(x, w):
    return jax.lax.dot_general(
        x, w, (((1,), (1,)), ((), ())), preferred_element_type=jnp.float32
    )


def _blockdiag(w, p):
    z = jnp.zeros_like(w)
    rows = [jnp.concatenate([w if j == i else z for j in range(p)], axis=1)
            for i in range(p)]
    return jnp.concatenate(rows, axis=0)


def _fwd_kernel(
    ap_feat_ref, a2u_feat_ref,
    wpa_ref, bpa_ref, wpe_ref, bpe_ref,
    w51_ref, b51_ref, w52_ref, b52_ref, w53_ref, b53_ref,
    w61_ref, b61_ref, w62_ref, b62_ref, w63_ref, b63_ref,
    w71_ref, b71_ref, w72_ref, b72_ref, w73_ref, b73_ref,
    wpo_ref, bpo_ref,
    out_ref,
):
    B, K, D, E, TA, P = _B, _K, _D, _E, _TA, _PACK
    DP = D * P

    ap_hid = _relu(_dot_t(ap_feat_ref[...], wpa_ref[...]) + bpa_ref[...])
    ea = _relu(_dot_t(a2u_feat_ref[...], wpe_ref[...]) + bpe_ref[...])

    w5a, w5b = w51_ref[...][:, :D], w51_ref[...][:, D:]
    w6a, w6b = w61_ref[...][:, :D], w61_ref[...][:, D:]
    w7a, w7b = w71_ref[...][:, :D], w71_ref[...][:, D:]
    w62blk = _blockdiag(w62_ref[...], P)
    w63blk = _blockdiag(w63_ref[...], P)
    b62p = jnp.concatenate([b62_ref[...]] * P, axis=1)
    b63p = jnp.concatenate([b63_ref[...]] * P, axis=1)

    for _ in range(2):
        p5 = _dot_t(ap_hid, w5a)
        h = _relu((_dot_t(ea, w5b) + b51_ref[...]).reshape(B, K, D)
                  + p5[:, None, :])
        h = _relu(_dot_t(h.reshape(E, D), w52_ref[...]) + b52_ref[...])
        z = _dot_t(h, w53_ref[...]).reshape(B, K, D)
        m1 = jnp.max(z, axis=1, keepdims=True)
        is_max = z == m1
        n_max = jnp.sum(jnp.where(is_max, 1.0, 0.0), axis=1, keepdims=True)
        m2 = jnp.max(jnp.where(is_max, _NEG, z), axis=1, keepdims=True)
        max_ap = _relu(jnp.where(is_max & (n_max == 1.0), m2, m1)
                       + b53_ref[...][:, None, :])

        p6 = _dot_t(ap_hid, w6a) + b61_ref[...]
        q6 = _dot_t(ea, w6b)
        q6p = jnp.concatenate([q6] * P, axis=1)

        lane_g = jax.lax.broadcasted_iota(jnp.int32, (1, 1, 1, DP), 3) // D
        r_ids = jax.lax.broadcasted_iota(jnp.int32, (TA, 1, 1, 1), 0)
        ap_idx = jax.lax.broadcasted_iota(jnp.int32, (1, B, 1, 1), 1)

        chunks = []
        for base in range(0, B, TA * P):
            ptp = jnp.concatenate(
                [p6[base + g * TA:base + (g + 1) * TA, :] for g in range(P)],
                axis=1)
            x = _relu(q6p[None, :, :] + ptp[:, None, :]).reshape(TA * E, DP)
            x = _relu(_dot_t(x, w62blk) + b62p)
            z = _dot_t(x, w63blk).reshape(TA, B, K, DP)
            excl = ap_idx == (r_ids + (base + lane_g * TA))
            mz = jnp.max(jnp.where(excl, _NEG, z), axis=1)
            mz = _relu(mz + b63p[:, None, :])
            for g in range(P):
                chunks.append(mz[:, :, g * D:(g + 1) * D])
        max_ue = jnp.concatenate(chunks, axis=0)

        agg = jnp.maximum(max_ap, max_ue).reshape(E, D)
        g = _relu(_dot_t(agg, w7a) + _dot_t(ea, w7b) + b71_ref[...])
        g = _relu(_dot_t(g, w72_ref[...]) + b72_ref[...])
        ea = _relu(_dot_t(g, w73_ref[...]) + b73_ref[...])

    out_ref[...] = _dot_t(ea, wpo_ref[...]) + bpo_ref[...]


@jax.jit
def _run(ap_feat, a2u_feat, *ws):
    return pl.pallas_call(
        _fwd_kernel,
        out_shape=jax.ShapeDtypeStruct((_E, 2), jnp.float32),
    )(ap_feat, a2u_feat, *ws)


def kernel(ap_feat, ue_feat, ap2ue_feat, ue2ap_feat, params):
    p = params
    r = lambda b: b.reshape(1, -1)
    ws = [
        p["pre_ap"]["W"], r(p["pre_ap"]["b"]),
        p["pre_edge"]["W"], r(p["pre_edge"]["b"]),
        p["m5"]["l1"]["W"], r(p["m5"]["l1"]["b"]),
        p["m5"]["l2"]["W"], r(p["m5"]["l2"]["b"]),
        p["m5"]["l3"]["W"], r(p["m5"]["l3"]["b"]),
        p["m6"]["l1"]["W"], r(p["m6"]["l1"]["b"]),
        p["m6"]["l2"]["W"], r(p["m6"]["l2"]["b"]),
        p["m6"]["l3"]["W"], r(p["m6"]["l3"]["b"]),
        p["m7"]["l1"]["W"], r(p["m7"]["l1"]["b"]),
        p["m7"]["l2"]["W"], r(p["m7"]["l2"]["b"]),
        p["m7"]["l3"]["W"], r(p["m7"]["l3"]["b"]),
        p["post"]["W"], r(p["post"]["b"]),
    ]
    return _run(ap_feat, ap2ue_feat, *ws)


# speedup vs baseline: 1.0288x; 1.0288x over previous
import jax
import jax.numpy as jnp
from jax.experimental import pallas as pl

_B, _K, _D = 64, 32, 64
_E = _B * _K
_NEG = -1e30
_TA = 8
_PACK = 2


def _relu(v):
    return jnp.maximum(v, 0.0)


def _dot_t(x, w):
    return jax.lax.dot_general(
        x, w, (((1,), (1,)), ((), ())), preferred_element_type=jnp.float32
    )


def _blockdiag(w, p):
    z = jnp.zeros_like(w)
    rows = [jnp.concatenate([w if j == i else z for j in range(p)], axis=1)
            for i in range(p)]
    return jnp.concatenate(rows, axis=0)


def _fwd_kernel(
    ap_feat_ref, a2u_feat_ref,
    wpa_ref, bpa_ref, wpe_ref, bpe_ref,
    w51_ref, b51_ref, w52_ref, b52_ref, w53_ref, b53_ref,
    w61_ref, b61_ref, w62_ref, b62_ref, w63_ref, b63_ref,
    w71_ref, b71_ref, w72_ref, b72_ref, w73_ref, b73_ref,
    wpo_ref, bpo_ref,
    out_ref,
):
    B, K, D, E, TA, P = _B, _K, _D, _E, _TA, _PACK
    DP = D * P

    ap_hid = _relu(_dot_t(ap_feat_ref[...], wpa_ref[...]) + bpa_ref[...])
    ea = _relu(_dot_t(a2u_feat_ref[...], wpe_ref[...]) + bpe_ref[...])

    w5a, w5b = w51_ref[...][:, :D], w51_ref[...][:, D:]
    w6a, w6b = w61_ref[...][:, :D], w61_ref[...][:, D:]
    w7a, w7b = w71_ref[...][:, :D], w71_ref[...][:, D:]
    w62blk = _blockdiag(w62_ref[...], P)
    w63blk = _blockdiag(w63_ref[...], P)
    b62p = jnp.concatenate([b62_ref[...]] * P, axis=1)
    b63p = jnp.concatenate([b63_ref[...]] * P, axis=1)

    w56a = jnp.concatenate([w5a, w6a], axis=0)
    w56b = jnp.concatenate([w5b, w6b], axis=0)

    for _ in range(2):
        pp = _dot_t(ap_hid, w56a)
        qq = _dot_t(ea, w56b)
        p5 = pp[:, :D]
        h = _relu((qq[:, :D] + b51_ref[...]).reshape(B, K, D)
                  + p5[:, None, :])
        h = _relu(_dot_t(h.reshape(E, D), w52_ref[...]) + b52_ref[...])
        z = _dot_t(h, w53_ref[...]).reshape(B, K, D)
        m1 = jnp.max(z, axis=1, keepdims=True)
        is_max = z == m1
        n_max = jnp.sum(jnp.where(is_max, 1.0, 0.0), axis=1, keepdims=True)
        m2 = jnp.max(jnp.where(is_max, _NEG, z), axis=1, keepdims=True)
        max_ap = _relu(jnp.where(is_max & (n_max == 1.0), m2, m1)
                       + b53_ref[...][:, None, :])

        p6 = pp[:, D:] + b61_ref[...]
        q6 = qq[:, D:]
        q6p = jnp.concatenate([q6] * P, axis=1)

        lane_g = jax.lax.broadcasted_iota(jnp.int32, (1, 1, 1, DP), 3) // D
        r_ids = jax.lax.broadcasted_iota(jnp.int32, (TA, 1, 1, 1), 0)
        ap_idx = jax.lax.broadcasted_iota(jnp.int32, (1, B, 1, 1), 1)

        chunks = []
        for base in range(0, B, TA * P):
            ptp = jnp.concatenate(
                [p6[base + g * TA:base + (g + 1) * TA, :] for g in range(P)],
                axis=1)
            x = _relu(q6p[None, :, :] + ptp[:, None, :]).reshape(TA * E, DP)
            x = _relu(_dot_t(x, w62blk) + b62p)
            z = _dot_t(x, w63blk).reshape(TA, B, K, DP)
            excl = ap_idx == (r_ids + (base + lane_g * TA))
            mz = jnp.max(jnp.where(excl, _NEG, z), axis=1)
            mz = _relu(mz + b63p[:, None, :])
            for g in range(P):
                chunks.append(mz[:, :, g * D:(g + 1) * D])
        max_ue = jnp.concatenate(chunks, axis=0)

        agg = jnp.maximum(max_ap, max_ue).reshape(E, D)
        g = _relu(_dot_t(jnp.concatenate([agg, ea], axis=1), w71_ref[...])
                  + b71_ref[...])
        g = _relu(_dot_t(g, w72_ref[...]) + b72_ref[...])
        ea = _relu(_dot_t(g, w73_ref[...]) + b73_ref[...])

    out_ref[...] = _dot_t(ea, wpo_ref[...]) + bpo_ref[...]


@jax.jit
def _run(ap_feat, a2u_feat, *ws):
    return pl.pallas_call(
        _fwd_kernel,
        out_shape=jax.ShapeDtypeStruct((_E, 2), jnp.float32),
    )(ap_feat, a2u_feat, *ws)


def kernel(ap_feat, ue_feat, ap2ue_feat, ue2ap_feat, params):
    p = params
    r = lambda b: b.reshape(1, -1)
    ws = [
        p["pre_ap"]["W"], r(p["pre_ap"]["b"]),
        p["pre_edge"]["W"], r(p["pre_edge"]["b"]),
        p["m5"]["l1"]["W"], r(p["m5"]["l1"]["b"]),
        p["m5"]["l2"]["W"], r(p["m5"]["l2"]["b"]),
        p["m5"]["l3"]["W"], r(p["m5"]["l3"]["b"]),
        p["m6"]["l1"]["W"], r(p["m6"]["l1"]["b"]),
        p["m6"]["l2"]["W"], r(p["m6"]["l2"]["b"]),
        p["m6"]["l3"]["W"], r(p["m6"]["l3"]["b"]),
        p["m7"]["l1"]["W"], r(p["m7"]["l1"]["b"]),
        p["m7"]["l2"]["W"], r(p["m7"]["l2"]["b"]),
        p["m7"]["l3"]["W"], r(p["m7"]["l3"]["b"]),
        p["post"]["W"], r(p["post"]["b"]),
    ]
    return _run(ap_feat, ap2ue_feat, *ws)
